# Initial kernel scaffold; baseline (speedup 1.0000x reference)
import jax
import jax.numpy as jnp
from jax.experimental import pallas as pl


def kernel(x, edge_index, edge_attr, batch, num_graphs, W1, as1, ad1, We1, ae1, b1, W2, as2, ad2, We2, ae2, b2, W3, as3, ad3, We3, ae3, b3, fcW1, fcb1, fcW2, fcb2):
    raise NotImplementedError("write your pallas kernel here")


# speedup vs baseline: 27.4061x; 27.4061x over previous
import jax
import jax.numpy as jnp
from jax import lax
from jax.experimental import pallas as pl
from jax.experimental.pallas import tpu as pltpu
from jax.experimental.pallas import tpu_sc as plsc

N = 10000
E = 320000
G = 32
IN = 128
HID = 16
HEADS = 8
HC = HEADS * HID
AG = 5

NP = 10112
K = 64
EPT = NP
NCH = EPT // K
E_PAD = 32 * EPT
ROWW = 144
RPT = NP // 16

_f32 = jnp.float32
_i32 = jnp.int32


def _bcast16(v, idx):
  return lax.gather(
      v, idx[:, None],
      dimension_numbers=lax.GatherDimensionNumbers(
          offset_dims=(), collapsed_slice_dims=(0,), start_index_map=(0,)),
      slice_sizes=(1,),
      mode=lax.GatherScatterMode.PROMISE_IN_BOUNDS)


def _edge_body(src_hbm, dst_hbm, ea_hbm, nf_hbm, xt_hbm, cv_hbm, out_hbm,
               accum, srcb, dstb, eab, nfsb, nfdb, xtb, rows, cvb):
  c = lax.axis_index("c")
  s = lax.axis_index("s")
  wid = c * 16 + s

  def _zf(i, _):
    for j in range(ROWW // 16):
      rows[i, pl.ds(j * 16, 16)] = jnp.zeros((16,), _f32)
    return 0
  lax.fori_loop(0, K, _zf, 0)
  base = s * RPT
  nfull = RPT // K
  for k in range(nfull):
    pltpu.sync_copy(rows, accum.at[pl.ds(base + k * K, K)])
  rem = RPT - nfull * K
  if rem:
    pltpu.sync_copy(rows.at[pl.ds(0, rem)],
                    accum.at[pl.ds(base + nfull * K, rem)])
  pltpu.sync_copy(cv_hbm, cvb)
  plsc.subcore_barrier()

  def _chunk(g, _c):
    off = wid * EPT + g * K
    pltpu.sync_copy(src_hbm.at[pl.ds(off, K)], srcb.at[0])
    pltpu.sync_copy(dst_hbm.at[pl.ds(off, K)], dstb.at[0])
    pltpu.sync_copy(ea_hbm.at[pl.ds(off, K)], eab)
    pltpu.sync_copy(nf_hbm.at[srcb.at[0]], nfsb)
    pltpu.sync_copy(nf_hbm.at[dstb.at[0]], nfdb)
    pltpu.sync_copy(xt_hbm.at[srcb.at[0]], xtb)

    def _edge(e, _):
      nfs = nfsb[e]
      nfd = nfdb[e]
      iota_l = lax.iota(_i32, 16)
      rot_l = lax.rem(iota_l + 8, jnp.full((16,), 16, _i32))
      eav = _bcast16(eab[...], jnp.full((16,), e, _i32))
      alpha = nfs + _bcast16(nfd, rot_l) + eav * cvb[...]
      alpha = jnp.where(alpha > 0, alpha, 0.2 * alpha)
      p = jnp.exp(alpha)
      rows[e, pl.ds(128, 16)] = (
          jnp.where(iota_l < 8, p, 0.0)
          + jnp.where(iota_l == 8, eav, 0.0)
          + jnp.where(iota_l == 9, 1.0, 0.0))
      for h in range(HEADS):
        ph = _bcast16(p, jnp.full((16,), h, _i32))
        rows[e, pl.ds(h * HID, HID)] = xtb[e, h] * ph
      return 0
    lax.fori_loop(0, K, _edge, 0)

    pltpu.sync_copy(rows, accum.at[dstb.at[0]], add=True)
    return 0
  lax.fori_loop(0, NCH, _chunk, 0)

  plsc.subcore_barrier()
  for k in range(nfull):
    pltpu.sync_copy(accum.at[pl.ds(base + k * K, K)],
                    out_hbm.at[c, pl.ds(base + k * K, K)])
  if rem:
    pltpu.sync_copy(accum.at[pl.ds(base + nfull * K, rem)],
                    out_hbm.at[c, pl.ds(base + nfull * K, rem)])


_edge_call = pl.kernel(
    _edge_body,
    mesh=plsc.VectorSubcoreMesh(core_axis_name="c", subcore_axis_name="s"),
    compiler_params=pltpu.CompilerParams(use_tc_tiling_on_sc=False),
    out_type=jax.ShapeDtypeStruct((2, NP, ROWW), _f32),
    scratch_types=[
        pltpu.VMEM_SHARED((NP, ROWW), _f32),
        pltpu.VMEM((1, K), _i32),
        pltpu.VMEM((1, K), _i32),
        pltpu.VMEM((K,), _f32),
        pltpu.VMEM((K, 16), _f32),
        pltpu.VMEM((K, 16), _f32),
        pltpu.VMEM((K, HEADS, HID), _f32),
        pltpu.VMEM((K, ROWW), _f32),
        pltpu.VMEM((16,), _f32),
    ],
)


def _a_body(h_ref, w_ref, asf_ref, adf_ref, sa_ref, sd_ref, xt_ref, nf_ref):
  h = h_ref[...]
  xt = jnp.dot(h, w_ref[...], preferred_element_type=_f32, precision=lax.Precision.HIGHEST)
  xt_ref[...] = xt
  nf_ref[...] = (
      jnp.dot(xt * asf_ref[...], sa_ref[...], preferred_element_type=_f32, precision=lax.Precision.HIGHEST)
      + jnp.dot(xt * adf_ref[...], sd_ref[...], preferred_element_type=_f32, precision=lax.Precision.HIGHEST))


def _call_a(h, W, asf, adf, sa, sd):
  return pl.pallas_call(
      _a_body,
      out_shape=[jax.ShapeDtypeStruct((NP, HC), _f32),
                 jax.ShapeDtypeStruct((NP, 16), _f32)],
  )(h, W, asf, adf, sa, sd)


def _c_body(o0_ref, o1_ref, xt_ref, nf_ref, loop_ref, c8_ref, b_ref,
            r8_ref, mo_ref, md_ref, ms_ref, mc_ref, mn_ref, sl_ref,
            h_ref, loop_out_ref):
  acc = o0_ref[...] + o1_ref[...]
  outun = jnp.dot(acc, mo_ref[...], preferred_element_type=_f32, precision=lax.Precision.HIGHEST)
  den = jnp.dot(acc, md_ref[...], preferred_element_type=_f32, precision=lax.Precision.HIGHEST)
  sea = jnp.dot(acc, ms_ref[...], preferred_element_type=_f32, precision=lax.Precision.HIGHEST)
  cnt = jnp.dot(acc, mc_ref[...], preferred_element_type=_f32, precision=lax.Precision.HIGHEST)
  loop_out_ref[...] = sea / jnp.maximum(cnt, 1.0)
  r8 = r8_ref[...]
  sl = sl_ref[0, 0]
  a = (jnp.dot(nf_ref[...], mn_ref[...], preferred_element_type=_f32, precision=lax.Precision.HIGHEST)
       + loop_ref[...] * c8_ref[...])
  selfp = jnp.exp(jnp.where(a > 0, a, 0.2 * a)) * sl
  recip = 1.0 / (den + selfp + 1e-16)
  outun = outun + jnp.dot(selfp, r8, preferred_element_type=_f32, precision=lax.Precision.HIGHEST) * xt_ref[...]
  out = outun * jnp.dot(recip, r8, preferred_element_type=_f32, precision=lax.Precision.HIGHEST)
  h_ref[...] = jnp.maximum(out + b_ref[...], 0.0)


_CBR = NP // 4


def _call_c(o0, o1, xt, nf, loopv, c8, b, r8, mo, md, ms, mc, mn, sl):
  def row(w):
    return pl.BlockSpec((_CBR, w), lambda i: (i, 0))

  def full(a):
    return pl.BlockSpec(a.shape, lambda i: (0,) * a.ndim)

  return pl.pallas_call(
      _c_body,
      grid=(4,),
      in_specs=[row(ROWW), row(ROWW), row(HC), row(16), row(8),
                full(c8), full(b), full(r8), full(mo), full(md),
                full(ms), full(mc), full(mn), full(sl)],
      out_specs=[row(HC), row(8)],
      out_shape=[jax.ShapeDtypeStruct((NP, HC), _f32),
                 jax.ShapeDtypeStruct((NP, 8), _f32)],
  )(o0, o1, xt, nf, loopv, c8, b, r8, mo, md, ms, mc, mn, sl)


def _f_body(h_ref, batch_ref, rep_ref, offs_ref, w1a_ref, w1b_ref, b1_ref,
            w2_ref, b2_ref, out_ref):
  h = h_ref[...]
  bi = batch_ref[...]
  gi = lax.broadcasted_iota(_i32, (G, NP), 0)
  eq = (gi == bi).astype(_f32)
  lt = (bi < gi).astype(_f32)
  gsum = jnp.dot(eq, h, preferred_element_type=_f32, precision=lax.Precision.HIGHEST)
  cnt = jnp.sum(eq, axis=1, keepdims=True)
  starts = jnp.sum(lt, axis=1, keepdims=True)
  gemb = gsum / jnp.maximum(cnt, 1.0)
  rep = rep_ref[...]
  grep = jnp.dot(rep, gemb, preferred_element_type=_f32, precision=lax.Precision.HIGHEST)
  st160 = jnp.dot(rep, jnp.broadcast_to(starts, (G, 128)),
                  preferred_element_type=_f32, precision=lax.Precision.HIGHEST)[:, :1]
  idx = jnp.minimum(st160 + offs_ref[...], float(N - 1)).astype(_i32)
  ni = lax.broadcasted_iota(_i32, (G * AG, NP), 1)
  aoh = (ni == idx).astype(_f32)
  aemb = jnp.dot(aoh, h, preferred_element_type=_f32, precision=lax.Precision.HIGHEST)
  z = (jnp.dot(aemb, w1a_ref[...], preferred_element_type=_f32, precision=lax.Precision.HIGHEST)
       + jnp.dot(grep, w1b_ref[...], preferred_element_type=_f32, precision=lax.Precision.HIGHEST)
       + b1_ref[...])
  z = jnp.maximum(z, 0.0)
  out_ref[...] = (jnp.dot(z, w2_ref[...], preferred_element_type=_f32, precision=lax.Precision.HIGHEST)
                  + b2_ref[...])


def _call_f(h, batch2d, rep, offs, w1a, w1b, b1, w2, b2):
  return pl.pallas_call(
      _f_body,
      out_shape=jax.ShapeDtypeStruct((G * AG, 128), _f32),
  )(h, batch2d, rep, offs, w1a, w1b, b1, w2, b2)


def _cvec(We, ae):
  return (We.reshape(1, HEADS, HID) * ae).sum(-1).reshape(1, HEADS)


def kernel(x, edge_index, edge_attr, batch, num_graphs, W1, as1, ad1, We1,
           ae1, b1, W2, as2, ad2, We2, ae2, b2, W3, as3, ad3, We3, ae3, b3,
           fcW1, fcb1, fcW2, fcb2):
  src = edge_index[0]
  dst = edge_index[1]
  pad_e = E_PAD - E
  srcp = jnp.concatenate([src, jnp.full((pad_e,), N, _i32)])
  dstp = jnp.concatenate([dst, jnp.full((pad_e,), N, _i32)])
  eap = jnp.concatenate([edge_attr[:, 0], jnp.zeros((pad_e,), _f32)])
  xp = jnp.concatenate([x, jnp.zeros((NP - N, IN), _f32)])

  hsel = jnp.arange(HC, dtype=_i32) // HID
  hid8 = jnp.arange(HEADS, dtype=_i32)
  oh = (hsel[:, None] == hid8[None, :]).astype(_f32)
  sa = jnp.concatenate([oh, jnp.zeros((HC, 8), _f32)], axis=1)
  sd = jnp.concatenate([jnp.zeros((HC, 8), _f32), oh], axis=1)
  r8 = oh.T
  mo = jnp.concatenate([jnp.eye(HC, dtype=_f32),
                        jnp.zeros((ROWW - HC, HC), _f32)], axis=0)
  ar8 = jnp.arange(8)
  md = jnp.zeros((ROWW, 8), _f32).at[128 + ar8, ar8].set(1.0)
  ms = jnp.zeros((ROWW, 8), _f32).at[136, :].set(1.0)
  mc = jnp.zeros((ROWW, 8), _f32).at[137, :].set(1.0)
  mn = jnp.concatenate([jnp.eye(8, dtype=_f32),
                        jnp.eye(8, dtype=_f32)], axis=0)
  rep = (jnp.arange(G * AG, dtype=_i32)[:, None] // AG
         == jnp.arange(G, dtype=_i32)[None, :]).astype(_f32)
  offs = (jnp.arange(G * AG, dtype=_i32) % AG).astype(_f32)[:, None]
  batch2d = jnp.concatenate([batch, jnp.full((NP - N,), 1000, _i32)])
  batch2d = batch2d.reshape(1, NP)
  w1a = jnp.pad(fcW1[:HC], ((0, 0), (0, 128 - fcW1.shape[1])))
  w1b = jnp.pad(fcW1[HC:], ((0, 0), (0, 128 - fcW1.shape[1])))
  b1p = jnp.pad(fcb1, (0, 128 - fcb1.shape[0])).reshape(1, 128)
  w2p = jnp.pad(fcW2, ((0, 128 - fcW2.shape[0]), (0, 128 - fcW2.shape[1])))
  b2p = jnp.pad(fcb2, (0, 128 - fcb2.shape[0])).reshape(1, 128)

  layers = [
      (W1, as1, ad1, We1, ae1, b1, 0.0),
      (W2, as2, ad2, We2, ae2, b2, 1.0),
      (W3, as3, ad3, We3, ae3, b3, 1.0),
  ]

  h = xp
  loopv = jnp.zeros((NP, 8), _f32)
  for (W, a_s, a_d, We, a_e, b, sl) in layers:
    asf = a_s.reshape(1, HC)
    adf = a_d.reshape(1, HC)
    c8 = _cvec(We, a_e)
    cv16 = jnp.concatenate([c8[0], c8[0]])
    bp = b.reshape(1, HC)
    slv = jnp.full((1, 1), sl, _f32)
    xt, nf = _call_a(h, W, asf, adf, sa, sd)
    parts = _edge_call(srcp, dstp, eap, nf,
                       xt.reshape(NP, HEADS, HID), cv16)
    h, loopv = _call_c(parts[0], parts[1], xt, nf, loopv, c8, bp, r8,
                       mo, md, ms, mc, mn, slv)

  pred = _call_f(h, batch2d, rep, offs, w1a, w1b, b1p, w2p, b2p)
  return pred[:, :2].reshape(G, AG, 2)


# speedup vs baseline: 30.8927x; 1.1272x over previous
import jax
import jax.numpy as jnp
from jax import lax
from jax.experimental import pallas as pl
from jax.experimental.pallas import tpu as pltpu
from jax.experimental.pallas import tpu_sc as plsc

N = 10000
E = 320000
G = 32
IN = 128
HID = 16
HEADS = 8
HC = HEADS * HID
AG = 5

NP = 10112
K = 128
EPT = NP
NCH = EPT // K
E_PAD = 32 * EPT
ROWW = 144
RPT = NP // 16

_f32 = jnp.float32
_i32 = jnp.int32


def _bcast16(v, idx):
  return lax.gather(
      v, idx[:, None],
      dimension_numbers=lax.GatherDimensionNumbers(
          offset_dims=(), collapsed_slice_dims=(0,), start_index_map=(0,)),
      slice_sizes=(1,),
      mode=lax.GatherScatterMode.PROMISE_IN_BOUNDS)


def _edge_body(src_hbm, dst_hbm, ea_hbm, nf_hbm, xt_hbm, cv_hbm, out_hbm,
               accum, srcb, dstb, eab, nfsb, nfdb, xtb, rows, cvb):
  c = lax.axis_index("c")
  s = lax.axis_index("s")
  wid = c * 16 + s

  def _zf(i, _):
    for j in range(ROWW // 16):
      rows[i, pl.ds(j * 16, 16)] = jnp.zeros((16,), _f32)
    return 0
  lax.fori_loop(0, K, _zf, 0)
  base = s * RPT
  nfull = RPT // K
  for k in range(nfull):
    pltpu.sync_copy(rows, accum.at[pl.ds(base + k * K, K)])
  rem = RPT - nfull * K
  if rem:
    pltpu.sync_copy(rows.at[pl.ds(0, rem)],
                    accum.at[pl.ds(base + nfull * K, rem)])
  pltpu.sync_copy(cv_hbm, cvb)
  plsc.subcore_barrier()

  def _chunk(g, _c):
    off = wid * EPT + g * K
    pltpu.sync_copy(src_hbm.at[pl.ds(off, K)], srcb.at[0])
    pltpu.sync_copy(dst_hbm.at[pl.ds(off, K)], dstb.at[0])
    pltpu.sync_copy(ea_hbm.at[pl.ds(off, K)], eab)
    pltpu.sync_copy(nf_hbm.at[srcb.at[0]], nfsb)
    pltpu.sync_copy(nf_hbm.at[dstb.at[0]], nfdb)
    pltpu.sync_copy(xt_hbm.at[srcb.at[0]], xtb)

    def _edge(e, _):
      nfs = nfsb[e]
      nfd = nfdb[e]
      iota_l = lax.iota(_i32, 16)
      rot_l = lax.rem(iota_l + 8, jnp.full((16,), 16, _i32))
      eav = _bcast16(eab[...], jnp.full((16,), e, _i32))
      alpha = nfs + _bcast16(nfd, rot_l) + eav * cvb[...]
      alpha = jnp.where(alpha > 0, alpha, 0.2 * alpha)
      p = jnp.exp(alpha)
      rows[e, pl.ds(128, 16)] = (
          jnp.where(iota_l < 8, p, 0.0)
          + jnp.where(iota_l == 8, eav, 0.0)
          + jnp.where(iota_l == 9, 1.0, 0.0))
      for h in range(HEADS):
        ph = _bcast16(p, jnp.full((16,), h, _i32))
        rows[e, pl.ds(h * HID, HID)] = xtb[e, h] * ph
      return 0
    lax.fori_loop(0, K, _edge, 0)

    pltpu.sync_copy(rows, accum.at[dstb.at[0]], add=True)
    return 0
  lax.fori_loop(0, NCH, _chunk, 0)

  plsc.subcore_barrier()
  for k in range(nfull):
    pltpu.sync_copy(accum.at[pl.ds(base + k * K, K)],
                    out_hbm.at[c, pl.ds(base + k * K, K)])
  if rem:
    pltpu.sync_copy(accum.at[pl.ds(base + nfull * K, rem)],
                    out_hbm.at[c, pl.ds(base + nfull * K, rem)])


_edge_call = pl.kernel(
    _edge_body,
    mesh=plsc.VectorSubcoreMesh(core_axis_name="c", subcore_axis_name="s"),
    compiler_params=pltpu.CompilerParams(use_tc_tiling_on_sc=False),
    out_type=jax.ShapeDtypeStruct((2, NP, ROWW), _f32),
    scratch_types=[
        pltpu.VMEM_SHARED((NP, ROWW), _f32),
        pltpu.VMEM((1, K), _i32),
        pltpu.VMEM((1, K), _i32),
        pltpu.VMEM((K,), _f32),
        pltpu.VMEM((K, 16), _f32),
        pltpu.VMEM((K, 16), _f32),
        pltpu.VMEM((K, HEADS, HID), _f32),
        pltpu.VMEM((K, ROWW), _f32),
        pltpu.VMEM((16,), _f32),
    ],
)


def _a_body(h_ref, w_ref, asf_ref, adf_ref, sa_ref, sd_ref, xt_ref, nf_ref):
  h = h_ref[...]
  xt = jnp.dot(h, w_ref[...], preferred_element_type=_f32, precision=lax.Precision.HIGHEST)
  xt_ref[...] = xt
  nf_ref[...] = (
      jnp.dot(xt * asf_ref[...], sa_ref[...], preferred_element_type=_f32, precision=lax.Precision.HIGHEST)
      + jnp.dot(xt * adf_ref[...], sd_ref[...], preferred_element_type=_f32, precision=lax.Precision.HIGHEST))


def _call_a(h, W, asf, adf, sa, sd):
  return pl.pallas_call(
      _a_body,
      out_shape=[jax.ShapeDtypeStruct((NP, HC), _f32),
                 jax.ShapeDtypeStruct((NP, 16), _f32)],
  )(h, W, asf, adf, sa, sd)


def _c_body(o0_ref, o1_ref, xt_ref, nf_ref, loop_ref, c8_ref, b_ref,
            r8_ref, mo_ref, md_ref, ms_ref, mc_ref, mn_ref, sl_ref,
            h_ref, loop_out_ref):
  acc = o0_ref[...] + o1_ref[...]
  outun = jnp.dot(acc, mo_ref[...], preferred_element_type=_f32, precision=lax.Precision.HIGHEST)
  den = jnp.dot(acc, md_ref[...], preferred_element_type=_f32, precision=lax.Precision.HIGHEST)
  sea = jnp.dot(acc, ms_ref[...], preferred_element_type=_f32, precision=lax.Precision.HIGHEST)
  cnt = jnp.dot(acc, mc_ref[...], preferred_element_type=_f32, precision=lax.Precision.HIGHEST)
  loop_out_ref[...] = sea / jnp.maximum(cnt, 1.0)
  r8 = r8_ref[...]
  sl = sl_ref[0, 0]
  a = (jnp.dot(nf_ref[...], mn_ref[...], preferred_element_type=_f32, precision=lax.Precision.HIGHEST)
       + loop_ref[...] * c8_ref[...])
  selfp = jnp.exp(jnp.where(a > 0, a, 0.2 * a)) * sl
  recip = 1.0 / (den + selfp + 1e-16)
  outun = outun + jnp.dot(selfp, r8, preferred_element_type=_f32, precision=lax.Precision.HIGHEST) * xt_ref[...]
  out = outun * jnp.dot(recip, r8, preferred_element_type=_f32, precision=lax.Precision.HIGHEST)
  h_ref[...] = jnp.maximum(out + b_ref[...], 0.0)


_CBR = NP // 4


def _call_c(o0, o1, xt, nf, loopv, c8, b, r8, mo, md, ms, mc, mn, sl):
  def row(w):
    return pl.BlockSpec((_CBR, w), lambda i: (i, 0))

  def full(a):
    return pl.BlockSpec(a.shape, lambda i: (0,) * a.ndim)

  return pl.pallas_call(
      _c_body,
      grid=(4,),
      in_specs=[row(ROWW), row(ROWW), row(HC), row(16), row(8),
                full(c8), full(b), full(r8), full(mo), full(md),
                full(ms), full(mc), full(mn), full(sl)],
      out_specs=[row(HC), row(8)],
      out_shape=[jax.ShapeDtypeStruct((NP, HC), _f32),
                 jax.ShapeDtypeStruct((NP, 8), _f32)],
  )(o0, o1, xt, nf, loopv, c8, b, r8, mo, md, ms, mc, mn, sl)


def _f_body(h_ref, batch_ref, rep_ref, offs_ref, w1a_ref, w1b_ref, b1_ref,
            w2_ref, b2_ref, out_ref):
  h = h_ref[...]
  bi = batch_ref[...]
  gi = lax.broadcasted_iota(_i32, (G, NP), 0)
  eq = (gi == bi).astype(_f32)
  lt = (bi < gi).astype(_f32)
  gsum = jnp.dot(eq, h, preferred_element_type=_f32, precision=lax.Precision.HIGHEST)
  cnt = jnp.sum(eq, axis=1, keepdims=True)
  starts = jnp.sum(lt, axis=1, keepdims=True)
  gemb = gsum / jnp.maximum(cnt, 1.0)
  rep = rep_ref[...]
  grep = jnp.dot(rep, gemb, preferred_element_type=_f32, precision=lax.Precision.HIGHEST)
  st160 = jnp.dot(rep, jnp.broadcast_to(starts, (G, 128)),
                  preferred_element_type=_f32, precision=lax.Precision.HIGHEST)[:, :1]
  idx = jnp.minimum(st160 + offs_ref[...], float(N - 1)).astype(_i32)
  ni = lax.broadcasted_iota(_i32, (G * AG, NP), 1)
  aoh = (ni == idx).astype(_f32)
  aemb = jnp.dot(aoh, h, preferred_element_type=_f32, precision=lax.Precision.HIGHEST)
  z = (jnp.dot(aemb, w1a_ref[...], preferred_element_type=_f32, precision=lax.Precision.HIGHEST)
       + jnp.dot(grep, w1b_ref[...], preferred_element_type=_f32, precision=lax.Precision.HIGHEST)
       + b1_ref[...])
  z = jnp.maximum(z, 0.0)
  out_ref[...] = (jnp.dot(z, w2_ref[...], preferred_element_type=_f32, precision=lax.Precision.HIGHEST)
                  + b2_ref[...])


def _call_f(h, batch2d, rep, offs, w1a, w1b, b1, w2, b2):
  return pl.pallas_call(
      _f_body,
      out_shape=jax.ShapeDtypeStruct((G * AG, 128), _f32),
  )(h, batch2d, rep, offs, w1a, w1b, b1, w2, b2)


def _cvec(We, ae):
  return (We.reshape(1, HEADS, HID) * ae).sum(-1).reshape(1, HEADS)


def kernel(x, edge_index, edge_attr, batch, num_graphs, W1, as1, ad1, We1,
           ae1, b1, W2, as2, ad2, We2, ae2, b2, W3, as3, ad3, We3, ae3, b3,
           fcW1, fcb1, fcW2, fcb2):
  src = edge_index[0]
  dst = edge_index[1]
  pad_e = E_PAD - E
  srcp = jnp.concatenate([src, jnp.full((pad_e,), N, _i32)])
  dstp = jnp.concatenate([dst, jnp.full((pad_e,), N, _i32)])
  eap = jnp.concatenate([edge_attr[:, 0], jnp.zeros((pad_e,), _f32)])
  xp = jnp.concatenate([x, jnp.zeros((NP - N, IN), _f32)])

  hsel = jnp.arange(HC, dtype=_i32) // HID
  hid8 = jnp.arange(HEADS, dtype=_i32)
  oh = (hsel[:, None] == hid8[None, :]).astype(_f32)
  sa = jnp.concatenate([oh, jnp.zeros((HC, 8), _f32)], axis=1)
  sd = jnp.concatenate([jnp.zeros((HC, 8), _f32), oh], axis=1)
  r8 = oh.T
  mo = jnp.concatenate([jnp.eye(HC, dtype=_f32),
                        jnp.zeros((ROWW - HC, HC), _f32)], axis=0)
  ar8 = jnp.arange(8)
  md = jnp.zeros((ROWW, 8), _f32).at[128 + ar8, ar8].set(1.0)
  ms = jnp.zeros((ROWW, 8), _f32).at[136, :].set(1.0)
  mc = jnp.zeros((ROWW, 8), _f32).at[137, :].set(1.0)
  mn = jnp.concatenate([jnp.eye(8, dtype=_f32),
                        jnp.eye(8, dtype=_f32)], axis=0)
  rep = (jnp.arange(G * AG, dtype=_i32)[:, None] // AG
         == jnp.arange(G, dtype=_i32)[None, :]).astype(_f32)
  offs = (jnp.arange(G * AG, dtype=_i32) % AG).astype(_f32)[:, None]
  batch2d = jnp.concatenate([batch, jnp.full((NP - N,), 1000, _i32)])
  batch2d = batch2d.reshape(1, NP)
  w1a = jnp.pad(fcW1[:HC], ((0, 0), (0, 128 - fcW1.shape[1])))
  w1b = jnp.pad(fcW1[HC:], ((0, 0), (0, 128 - fcW1.shape[1])))
  b1p = jnp.pad(fcb1, (0, 128 - fcb1.shape[0])).reshape(1, 128)
  w2p = jnp.pad(fcW2, ((0, 128 - fcW2.shape[0]), (0, 128 - fcW2.shape[1])))
  b2p = jnp.pad(fcb2, (0, 128 - fcb2.shape[0])).reshape(1, 128)

  layers = [
      (W1, as1, ad1, We1, ae1, b1, 0.0),
      (W2, as2, ad2, We2, ae2, b2, 1.0),
      (W3, as3, ad3, We3, ae3, b3, 1.0),
  ]

  h = xp
  loopv = jnp.zeros((NP, 8), _f32)
  for (W, a_s, a_d, We, a_e, b, sl) in layers:
    asf = a_s.reshape(1, HC)
    adf = a_d.reshape(1, HC)
    c8 = _cvec(We, a_e)
    cv16 = jnp.concatenate([c8[0], c8[0]])
    bp = b.reshape(1, HC)
    slv = jnp.full((1, 1), sl, _f32)
    xt, nf = _call_a(h, W, asf, adf, sa, sd)
    parts = _edge_call(srcp, dstp, eap, nf,
                       xt.reshape(NP, HEADS, HID), cv16)
    h, loopv = _call_c(parts[0], parts[1], xt, nf, loopv, c8, bp, r8,
                       mo, md, ms, mc, mn, slv)

  pred = _call_f(h, batch2d, rep, offs, w1a, w1b, b1p, w2p, b2p)
  return pred[:, :2].reshape(G, AG, 2)


# speedup vs baseline: 34.8799x; 1.1291x over previous
import jax
import jax.numpy as jnp
from jax import lax
from jax.experimental import pallas as pl
from jax.experimental.pallas import tpu as pltpu
from jax.experimental.pallas import tpu_sc as plsc

N = 10000
E = 320000
G = 32
IN = 128
HID = 16
HEADS = 8
HC = HEADS * HID
AG = 5

NP = 10112
K = 64
EPT = NP
NCH = EPT // K
E_PAD = 32 * EPT
ROWW = 144
RPT = NP // 16

_f32 = jnp.float32
_i32 = jnp.int32


def _bcast16(v, idx):
  return lax.gather(
      v, idx[:, None],
      dimension_numbers=lax.GatherDimensionNumbers(
          offset_dims=(), collapsed_slice_dims=(0,), start_index_map=(0,)),
      slice_sizes=(1,),
      mode=lax.GatherScatterMode.PROMISE_IN_BOUNDS)


def _edge_body(src_hbm, dst_hbm, ea_hbm, nf_hbm, xt_hbm, cv_hbm, out_hbm,
               accum, srcb, dstb, eab, nfsb, nfdb, xtb, rows, cvb,
               sem1, sem2, sem3):
  c = lax.axis_index("c")
  s = lax.axis_index("s")
  wid = c * 16 + s

  def _zf(i, _):
    for j in range(ROWW // 16):
      rows[i, pl.ds(j * 16, 16)] = jnp.zeros((16,), _f32)
    return 0
  lax.fori_loop(0, K, _zf, 0)
  base = s * RPT
  nfull = RPT // K
  for k in range(nfull):
    pltpu.sync_copy(rows, accum.at[pl.ds(base + k * K, K)])
  rem = RPT - nfull * K
  if rem:
    pltpu.sync_copy(rows.at[pl.ds(0, rem)],
                    accum.at[pl.ds(base + nfull * K, rem)])
  pltpu.sync_copy(cv_hbm, cvb)
  plsc.subcore_barrier()

  def _chunk(g, _c):
    off = wid * EPT + g * K
    a1 = pltpu.async_copy(src_hbm.at[pl.ds(off, K)], srcb.at[0], sem1)
    a2 = pltpu.async_copy(dst_hbm.at[pl.ds(off, K)], dstb.at[0], sem1)
    a3 = pltpu.async_copy(ea_hbm.at[pl.ds(off, K)], eab, sem1)
    a1.wait()
    a2.wait()
    a3.wait()
    b1 = pltpu.async_copy(nf_hbm.at[srcb.at[0]], nfsb, sem2)
    b2 = pltpu.async_copy(nf_hbm.at[dstb.at[0]], nfdb, sem2)
    b3 = pltpu.async_copy(xt_hbm.at[srcb.at[0]], xtb, sem3)
    b1.wait()
    b2.wait()
    b3.wait()

    def _edge(e, _):
      nfs = nfsb[e]
      nfd = nfdb[e]
      iota_l = lax.iota(_i32, 16)
      rot_l = lax.rem(iota_l + 8, jnp.full((16,), 16, _i32))
      eav = _bcast16(eab[...], jnp.full((16,), e, _i32))
      alpha = nfs + _bcast16(nfd, rot_l) + eav * cvb[...]
      alpha = jnp.where(alpha > 0, alpha, 0.2 * alpha)
      p = jnp.exp(alpha)
      rows[e, pl.ds(128, 16)] = (
          jnp.where(iota_l < 8, p, 0.0)
          + jnp.where(iota_l == 8, eav, 0.0)
          + jnp.where(iota_l == 9, 1.0, 0.0))
      for h in range(HEADS):
        ph = _bcast16(p, jnp.full((16,), h, _i32))
        rows[e, pl.ds(h * HID, HID)] = xtb[e, h] * ph
      return 0
    lax.fori_loop(0, K, _edge, 0)

    pltpu.sync_copy(rows, accum.at[dstb.at[0]], add=True)
    return 0
  lax.fori_loop(0, NCH, _chunk, 0)

  plsc.subcore_barrier()
  for k in range(nfull):
    pltpu.sync_copy(accum.at[pl.ds(base + k * K, K)],
                    out_hbm.at[c, pl.ds(base + k * K, K)])
  if rem:
    pltpu.sync_copy(accum.at[pl.ds(base + nfull * K, rem)],
                    out_hbm.at[c, pl.ds(base + nfull * K, rem)])


_edge_call = pl.kernel(
    _edge_body,
    mesh=plsc.VectorSubcoreMesh(core_axis_name="c", subcore_axis_name="s"),
    compiler_params=pltpu.CompilerParams(use_tc_tiling_on_sc=False),
    out_type=jax.ShapeDtypeStruct((2, NP, ROWW), _f32),
    scratch_types=[
        pltpu.VMEM_SHARED((NP, ROWW), _f32),
        pltpu.VMEM((1, K), _i32),
        pltpu.VMEM((1, K), _i32),
        pltpu.VMEM((K,), _f32),
        pltpu.VMEM((K, 16), _f32),
        pltpu.VMEM((K, 16), _f32),
        pltpu.VMEM((K, HEADS, HID), _f32),
        pltpu.VMEM((K, ROWW), _f32),
        pltpu.VMEM((16,), _f32),
        pltpu.SemaphoreType.DMA,
        pltpu.SemaphoreType.DMA,
        pltpu.SemaphoreType.DMA,
    ],
)


def _a_body(h_ref, w_ref, asf_ref, adf_ref, sa_ref, sd_ref, xt_ref, nf_ref):
  h = h_ref[...]
  xt = jnp.dot(h, w_ref[...], preferred_element_type=_f32, precision=lax.Precision.HIGHEST)
  xt_ref[...] = xt
  nf_ref[...] = (
      jnp.dot(xt * asf_ref[...], sa_ref[...], preferred_element_type=_f32, precision=lax.Precision.HIGHEST)
      + jnp.dot(xt * adf_ref[...], sd_ref[...], preferred_element_type=_f32, precision=lax.Precision.HIGHEST))


def _call_a(h, W, asf, adf, sa, sd):
  return pl.pallas_call(
      _a_body,
      out_shape=[jax.ShapeDtypeStruct((NP, HC), _f32),
                 jax.ShapeDtypeStruct((NP, 16), _f32)],
  )(h, W, asf, adf, sa, sd)


def _c_body(o0_ref, o1_ref, xt_ref, nf_ref, loop_ref, c8_ref, b_ref,
            r8_ref, mo_ref, md_ref, ms_ref, mc_ref, mn_ref, sl_ref,
            h_ref, loop_out_ref):
  acc = o0_ref[...] + o1_ref[...]
  outun = jnp.dot(acc, mo_ref[...], preferred_element_type=_f32, precision=lax.Precision.HIGHEST)
  den = jnp.dot(acc, md_ref[...], preferred_element_type=_f32, precision=lax.Precision.HIGHEST)
  sea = jnp.dot(acc, ms_ref[...], preferred_element_type=_f32, precision=lax.Precision.HIGHEST)
  cnt = jnp.dot(acc, mc_ref[...], preferred_element_type=_f32, precision=lax.Precision.HIGHEST)
  loop_out_ref[...] = sea / jnp.maximum(cnt, 1.0)
  r8 = r8_ref[...]
  sl = sl_ref[0, 0]
  a = (jnp.dot(nf_ref[...], mn_ref[...], preferred_element_type=_f32, precision=lax.Precision.HIGHEST)
       + loop_ref[...] * c8_ref[...])
  selfp = jnp.exp(jnp.where(a > 0, a, 0.2 * a)) * sl
  recip = 1.0 / (den + selfp + 1e-16)
  outun = outun + jnp.dot(selfp, r8, preferred_element_type=_f32, precision=lax.Precision.HIGHEST) * xt_ref[...]
  out = outun * jnp.dot(recip, r8, preferred_element_type=_f32, precision=lax.Precision.HIGHEST)
  h_ref[...] = jnp.maximum(out + b_ref[...], 0.0)


_CBR = NP // 4


def _call_c(o0, o1, xt, nf, loopv, c8, b, r8, mo, md, ms, mc, mn, sl):
  def row(w):
    return pl.BlockSpec((_CBR, w), lambda i: (i, 0))

  def full(a):
    return pl.BlockSpec(a.shape, lambda i: (0,) * a.ndim)

  return pl.pallas_call(
      _c_body,
      grid=(4,),
      in_specs=[row(ROWW), row(ROWW), row(HC), row(16), row(8),
                full(c8), full(b), full(r8), full(mo), full(md),
                full(ms), full(mc), full(mn), full(sl)],
      out_specs=[row(HC), row(8)],
      out_shape=[jax.ShapeDtypeStruct((NP, HC), _f32),
                 jax.ShapeDtypeStruct((NP, 8), _f32)],
  )(o0, o1, xt, nf, loopv, c8, b, r8, mo, md, ms, mc, mn, sl)


def _f_body(h_ref, batch_ref, rep_ref, offs_ref, w1a_ref, w1b_ref, b1_ref,
            w2_ref, b2_ref, out_ref):
  h = h_ref[...]
  bi = batch_ref[...]
  gi = lax.broadcasted_iota(_i32, (G, NP), 0)
  eq = (gi == bi).astype(_f32)
  lt = (bi < gi).astype(_f32)
  gsum = jnp.dot(eq, h, preferred_element_type=_f32, precision=lax.Precision.HIGHEST)
  cnt = jnp.sum(eq, axis=1, keepdims=True)
  starts = jnp.sum(lt, axis=1, keepdims=True)
  gemb = gsum / jnp.maximum(cnt, 1.0)
  rep = rep_ref[...]
  grep = jnp.dot(rep, gemb, preferred_element_type=_f32, precision=lax.Precision.HIGHEST)
  st160 = jnp.dot(rep, jnp.broadcast_to(starts, (G, 128)),
                  preferred_element_type=_f32, precision=lax.Precision.HIGHEST)[:, :1]
  idx = jnp.minimum(st160 + offs_ref[...], float(N - 1)).astype(_i32)
  ni = lax.broadcasted_iota(_i32, (G * AG, NP), 1)
  aoh = (ni == idx).astype(_f32)
  aemb = jnp.dot(aoh, h, preferred_element_type=_f32, precision=lax.Precision.HIGHEST)
  z = (jnp.dot(aemb, w1a_ref[...], preferred_element_type=_f32, precision=lax.Precision.HIGHEST)
       + jnp.dot(grep, w1b_ref[...], preferred_element_type=_f32, precision=lax.Precision.HIGHEST)
       + b1_ref[...])
  z = jnp.maximum(z, 0.0)
  out_ref[...] = (jnp.dot(z, w2_ref[...], preferred_element_type=_f32, precision=lax.Precision.HIGHEST)
                  + b2_ref[...])


def _call_f(h, batch2d, rep, offs, w1a, w1b, b1, w2, b2):
  return pl.pallas_call(
      _f_body,
      out_shape=jax.ShapeDtypeStruct((G * AG, 128), _f32),
  )(h, batch2d, rep, offs, w1a, w1b, b1, w2, b2)


def _cvec(We, ae):
  return (We.reshape(1, HEADS, HID) * ae).sum(-1).reshape(1, HEADS)


def kernel(x, edge_index, edge_attr, batch, num_graphs, W1, as1, ad1, We1,
           ae1, b1, W2, as2, ad2, We2, ae2, b2, W3, as3, ad3, We3, ae3, b3,
           fcW1, fcb1, fcW2, fcb2):
  src = edge_index[0]
  dst = edge_index[1]
  pad_e = E_PAD - E
  srcp = jnp.concatenate([src, jnp.full((pad_e,), N, _i32)])
  dstp = jnp.concatenate([dst, jnp.full((pad_e,), N, _i32)])
  eap = jnp.concatenate([edge_attr[:, 0], jnp.zeros((pad_e,), _f32)])
  xp = jnp.concatenate([x, jnp.zeros((NP - N, IN), _f32)])

  hsel = jnp.arange(HC, dtype=_i32) // HID
  hid8 = jnp.arange(HEADS, dtype=_i32)
  oh = (hsel[:, None] == hid8[None, :]).astype(_f32)
  sa = jnp.concatenate([oh, jnp.zeros((HC, 8), _f32)], axis=1)
  sd = jnp.concatenate([jnp.zeros((HC, 8), _f32), oh], axis=1)
  r8 = oh.T
  mo = jnp.concatenate([jnp.eye(HC, dtype=_f32),
                        jnp.zeros((ROWW - HC, HC), _f32)], axis=0)
  ar8 = jnp.arange(8)
  md = jnp.zeros((ROWW, 8), _f32).at[128 + ar8, ar8].set(1.0)
  ms = jnp.zeros((ROWW, 8), _f32).at[136, :].set(1.0)
  mc = jnp.zeros((ROWW, 8), _f32).at[137, :].set(1.0)
  mn = jnp.concatenate([jnp.eye(8, dtype=_f32),
                        jnp.eye(8, dtype=_f32)], axis=0)
  rep = (jnp.arange(G * AG, dtype=_i32)[:, None] // AG
         == jnp.arange(G, dtype=_i32)[None, :]).astype(_f32)
  offs = (jnp.arange(G * AG, dtype=_i32) % AG).astype(_f32)[:, None]
  batch2d = jnp.concatenate([batch, jnp.full((NP - N,), 1000, _i32)])
  batch2d = batch2d.reshape(1, NP)
  w1a = jnp.pad(fcW1[:HC], ((0, 0), (0, 128 - fcW1.shape[1])))
  w1b = jnp.pad(fcW1[HC:], ((0, 0), (0, 128 - fcW1.shape[1])))
  b1p = jnp.pad(fcb1, (0, 128 - fcb1.shape[0])).reshape(1, 128)
  w2p = jnp.pad(fcW2, ((0, 128 - fcW2.shape[0]), (0, 128 - fcW2.shape[1])))
  b2p = jnp.pad(fcb2, (0, 128 - fcb2.shape[0])).reshape(1, 128)

  layers = [
      (W1, as1, ad1, We1, ae1, b1, 0.0),
      (W2, as2, ad2, We2, ae2, b2, 1.0),
      (W3, as3, ad3, We3, ae3, b3, 1.0),
  ]

  h = xp
  loopv = jnp.zeros((NP, 8), _f32)
  for (W, a_s, a_d, We, a_e, b, sl) in layers:
    asf = a_s.reshape(1, HC)
    adf = a_d.reshape(1, HC)
    c8 = _cvec(We, a_e)
    cv16 = jnp.concatenate([c8[0], c8[0]])
    bp = b.reshape(1, HC)
    slv = jnp.full((1, 1), sl, _f32)
    xt, nf = _call_a(h, W, asf, adf, sa, sd)
    parts = _edge_call(srcp, dstp, eap, nf,
                       xt.reshape(NP, HEADS, HID), cv16)
    h, loopv = _call_c(parts[0], parts[1], xt, nf, loopv, c8, bp, r8,
                       mo, md, ms, mc, mn, slv)

  pred = _call_f(h, batch2d, rep, offs, w1a, w1b, b1p, w2p, b2p)
  return pred[:, :2].reshape(G, AG, 2)
